# Initial kernel scaffold; baseline (speedup 1.0000x reference)
import jax
import jax.numpy as jnp
from jax.experimental import pallas as pl


def kernel(nodes, neigh_idx, seq_length, num_sample, features_table, W_in, b_in):
    raise NotImplementedError("write your pallas kernel here")


# speedup vs baseline: 1.4371x; 1.4371x over previous
import functools

import jax
import jax.numpy as jnp
from jax import lax
from jax.experimental import pallas as pl
from jax.experimental.pallas import tpu as pltpu
from jax.experimental.pallas import tpu_sc as plsc


def _proj_body(shift_ref, feat_ref, w_ref, b_ref, seq_ref, out_ref, mask_ref):
    x = feat_ref[...].astype(jnp.bfloat16)
    w = w_ref[...].astype(jnp.bfloat16)
    acc = lax.dot_general(x, w, dimension_numbers=(((1,), (0,)), ((), ())),
                          preferred_element_type=jnp.float32)
    out_ref[...] = acc + b_ref[...]

    @pl.when(pl.program_id(0) == 0)
    def _():
        pos = lax.broadcasted_iota(jnp.int32, mask_ref.shape, 1) + 1 + shift_ref[0]
        mask_ref[...] = (pos > seq_ref[...]).astype(jnp.int8)


def _project_table(features_table, w, b2d, seq2d, shift, k):
    n, d = features_table.shape
    e = w.shape[1]
    bk = seq2d.shape[0]
    blk = 1024
    grid = (pl.cdiv(n, blk),)
    out_shape = [
        jax.ShapeDtypeStruct((n, e), jnp.float32),
        jax.ShapeDtypeStruct((bk, k), jnp.int8),
    ]
    return pl.pallas_call(
        _proj_body,
        grid=grid,
        in_specs=[
            pl.BlockSpec(memory_space=pltpu.SMEM),
            pl.BlockSpec((blk, d), lambda i: (i, 0)),
            pl.BlockSpec((d, e), lambda i: (0, 0)),
            pl.BlockSpec((1, e), lambda i: (0, 0)),
            pl.BlockSpec((bk, 1), lambda i: (0, 0)),
        ],
        out_specs=[
            pl.BlockSpec((blk, e), lambda i: (i, 0)),
            pl.BlockSpec((bk, k), lambda i: (0, 0)),
        ],
        out_shape=out_shape,
    )(shift, features_table, w, b2d, seq2d)


def _pick_chunk(per_w):
    for c in (128, 64, 32, 16, 8):
        if per_w % c == 0:
            return c
    return per_w


def _sc_gather(table, idx, n_rows, e, nc, ns):
    nw = nc * ns
    per_w = n_rows // nw
    chunk = _pick_chunk(per_w)
    steps = per_w // chunk
    mesh = plsc.VectorSubcoreMesh(core_axis_name="c", subcore_axis_name="s")

    @functools.partial(
        pl.kernel,
        mesh=mesh,
        out_type=jax.ShapeDtypeStruct((n_rows, e), jnp.float32),
        scratch_types=[
            pltpu.VMEM((chunk,), jnp.int32),
            pltpu.VMEM((chunk, e), jnp.float32),
            pltpu.SemaphoreType.DMA,
        ],
    )
    def gather_k(idx_hbm, tab_hbm, out_hbm, idx_v, rows_v, sem):
        wid = lax.axis_index("s") * nc + lax.axis_index("c")

        def one(i, carry):
            base = wid * per_w + i * chunk
            pltpu.sync_copy(idx_hbm.at[pl.ds(base, chunk)], idx_v)
            pltpu.async_copy(tab_hbm.at[idx_v], rows_v, sem).wait()
            pltpu.sync_copy(rows_v, out_hbm.at[pl.ds(base, chunk)])
            return carry

        if steps == 1:
            one(0, 0)
        else:
            lax.fori_loop(0, steps, one, 0)

    return gather_k(idx, table)


def kernel(nodes, neigh_idx, seq_length, num_sample, features_table, W_in, b_in):
    b, k = neigh_idx.shape
    n, d = features_table.shape
    e = W_in.shape[1]

    shift = (jnp.asarray(num_sample, jnp.int32) - k).reshape(1)
    seq2d = seq_length.astype(jnp.int32).reshape(b, 1)
    b2d = b_in.reshape(1, e)

    proj, mask_i8 = _project_table(features_table, W_in, b2d, seq2d, shift, k)

    try:
        info = plsc.get_sparse_core_info()
        nc, ns = info.num_cores, info.num_subcores
    except (RuntimeError, ValueError):
        nc, ns = 2, 16

    nodes_i = nodes.astype(jnp.int32)
    neigh_i = neigh_idx.astype(jnp.int32).reshape(-1)

    nodes_emb = _sc_gather(proj, nodes_i, b, e, nc, ns)
    neighs_flat = _sc_gather(proj, neigh_i, b * k, e, nc, ns)

    return (nodes_emb,
            neighs_flat.reshape(b, k, e),
            mask_i8.astype(jnp.bool_))


# speedup vs baseline: 1.4414x; 1.0029x over previous
import functools

import jax
import jax.numpy as jnp
from jax import lax
from jax.experimental import pallas as pl
from jax.experimental.pallas import tpu as pltpu
from jax.experimental.pallas import tpu_sc as plsc


def _proj_body(feat_ref, w_ref, b_ref, out_ref):
    x = feat_ref[...].astype(jnp.bfloat16)
    w = w_ref[...].astype(jnp.bfloat16)
    acc = lax.dot_general(x, w, dimension_numbers=(((1,), (0,)), ((), ())),
                          preferred_element_type=jnp.float32)
    out_ref[...] = acc + b_ref[...]


def _project_table(features_table, w, b2d):
    n, d = features_table.shape
    e = w.shape[1]
    blk = 1024
    grid = (pl.cdiv(n, blk),)
    return pl.pallas_call(
        _proj_body,
        grid=grid,
        in_specs=[
            pl.BlockSpec((blk, d), lambda i: (i, 0)),
            pl.BlockSpec((d, e), lambda i: (0, 0)),
            pl.BlockSpec((1, e), lambda i: (0, 0)),
        ],
        out_specs=pl.BlockSpec((blk, e), lambda i: (i, 0)),
        out_shape=jax.ShapeDtypeStruct((n, e), jnp.float32),
        compiler_params=pltpu.CompilerParams(
            dimension_semantics=("parallel",)),
    )(features_table, w, b2d)


def _mask_body(shift_ref, seq_ref, mask_ref):
    pos = lax.broadcasted_iota(jnp.int32, mask_ref.shape, 1) + 1 + shift_ref[0]
    mask_ref[...] = (pos > seq_ref[...]).astype(jnp.int8)


def _build_mask(seq2d, shift, k):
    bk = seq2d.shape[0]
    return pl.pallas_call(
        _mask_body,
        in_specs=[
            pl.BlockSpec(memory_space=pltpu.SMEM),
            pl.BlockSpec((bk, 1), lambda: (0, 0)),
        ],
        out_specs=pl.BlockSpec((bk, k), lambda: (0, 0)),
        out_shape=jax.ShapeDtypeStruct((bk, k), jnp.int8),
    )(shift, seq2d)


def _pick_chunk(per_w):
    for c in (128, 64, 32, 16, 8):
        if per_w % c == 0:
            return c
    return per_w


def _sc_gather(table, idx, n_rows, e, nc, ns):
    nw = nc * ns
    per_w = n_rows // nw
    chunk = _pick_chunk(per_w)
    steps = per_w // chunk
    mesh = plsc.VectorSubcoreMesh(core_axis_name="c", subcore_axis_name="s")

    @functools.partial(
        pl.kernel,
        mesh=mesh,
        out_type=jax.ShapeDtypeStruct((n_rows, e), jnp.float32),
        scratch_types=[
            pltpu.VMEM((chunk,), jnp.int32),
            pltpu.VMEM((chunk, e), jnp.float32),
            pltpu.SemaphoreType.DMA,
        ],
    )
    def gather_k(idx_hbm, tab_hbm, out_hbm, idx_v, rows_v, sem):
        wid = lax.axis_index("s") * nc + lax.axis_index("c")

        def one(i, carry):
            base = wid * per_w + i * chunk
            pltpu.sync_copy(idx_hbm.at[pl.ds(base, chunk)], idx_v)
            pltpu.async_copy(tab_hbm.at[idx_v], rows_v, sem).wait()
            pltpu.sync_copy(rows_v, out_hbm.at[pl.ds(base, chunk)])
            return carry

        if steps == 1:
            one(0, 0)
        else:
            lax.fori_loop(0, steps, one, 0)

    return gather_k(idx, table)


def kernel(nodes, neigh_idx, seq_length, num_sample, features_table, W_in, b_in):
    b, k = neigh_idx.shape
    n, d = features_table.shape
    e = W_in.shape[1]

    shift = (jnp.asarray(num_sample, jnp.int32) - k).reshape(1)
    seq2d = seq_length.astype(jnp.int32).reshape(b, 1)
    b2d = b_in.reshape(1, e)

    proj = _project_table(features_table, W_in, b2d)
    mask_i8 = _build_mask(seq2d, shift, k)

    try:
        info = plsc.get_sparse_core_info()
        nc, ns = info.num_cores, info.num_subcores
    except (RuntimeError, ValueError):
        nc, ns = 2, 16

    nodes_i = nodes.astype(jnp.int32)
    neigh_i = neigh_idx.astype(jnp.int32)

    nodes_emb = _sc_gather(proj, nodes_i, b, e, nc, ns)
    neighs_flat = _sc_gather(proj, neigh_i.reshape(-1), b * k, e, nc, ns)

    return (nodes_emb,
            neighs_flat.reshape(b, k, e),
            mask_i8.astype(jnp.bool_))


# speedup vs baseline: 1.4780x; 1.0254x over previous
import functools

import jax
import jax.numpy as jnp
from jax import lax
from jax.experimental import pallas as pl
from jax.experimental.pallas import tpu as pltpu
from jax.experimental.pallas import tpu_sc as plsc


def _proj_body(feat_ref, w_ref, b_ref, o0, o1, o2, o3):
    x = feat_ref[...].astype(jnp.bfloat16)
    w = w_ref[...].astype(jnp.bfloat16)
    acc = lax.dot_general(x, w, dimension_numbers=(((1,), (0,)), ((), ())),
                          preferred_element_type=jnp.float32)
    acc = acc + b_ref[...]
    o0[...] = acc[:, 0:128]
    o1[...] = acc[:, 128:256]
    o2[...] = acc[:, 256:384]
    o3[...] = acc[:, 384:512]


def _project_table(features_table, w, b2d):
    n, d = features_table.shape
    e = w.shape[1]
    blk = 1024
    grid = (pl.cdiv(n, blk),)
    part = jax.ShapeDtypeStruct((n, 128), jnp.float32)
    return pl.pallas_call(
        _proj_body,
        grid=grid,
        in_specs=[
            pl.BlockSpec((blk, d), lambda i: (i, 0)),
            pl.BlockSpec((d, e), lambda i: (0, 0)),
            pl.BlockSpec((1, e), lambda i: (0, 0)),
        ],
        out_specs=[pl.BlockSpec((blk, 128), lambda i: (i, 0))] * 4,
        out_shape=[part] * 4,
        compiler_params=pltpu.CompilerParams(
            dimension_semantics=("parallel",)),
    )(features_table, w, b2d)


def _pick_chunk(per_w):
    for c in (128, 64, 32, 16, 8):
        if per_w % c == 0:
            return c
    return per_w


def _sc_gather4(parts, idx, n_rows, nc, ns):
    nw = nc * ns
    per_w = n_rows // nw
    chunk = _pick_chunk(per_w)
    steps = per_w // chunk
    mesh = plsc.VectorSubcoreMesh(core_axis_name="c", subcore_axis_name="s")
    part = jax.ShapeDtypeStruct((n_rows, 128), jnp.float32)

    @functools.partial(
        pl.kernel,
        mesh=mesh,
        out_type=[part] * 4,
        scratch_types=[
            pltpu.VMEM((chunk,), jnp.int32),
            pltpu.VMEM((chunk, 128), jnp.float32),
            pltpu.VMEM((chunk, 128), jnp.float32),
            pltpu.VMEM((chunk, 128), jnp.float32),
            pltpu.VMEM((chunk, 128), jnp.float32),
            pltpu.SemaphoreType.DMA,
        ],
    )
    def gather_k(idx_hbm, t0, t1, t2, t3, u0, u1, u2, u3,
                 idx_v, r0, r1, r2, r3, sem):
        wid = lax.axis_index("s") * nc + lax.axis_index("c")
        tabs = (t0, t1, t2, t3)
        outs = (u0, u1, u2, u3)
        rows = (r0, r1, r2, r3)

        def one(i, carry):
            base = wid * per_w + i * chunk
            pltpu.sync_copy(idx_hbm.at[pl.ds(base, chunk)], idx_v)
            descs = [pltpu.async_copy(tabs[j].at[idx_v], rows[j], sem)
                     for j in range(4)]
            for dsc in descs:
                dsc.wait()
            for j in range(4):
                pltpu.sync_copy(rows[j], outs[j].at[pl.ds(base, chunk)])
            return carry

        if steps == 1:
            one(0, 0)
        else:
            lax.fori_loop(0, steps, one, 0)

    return gather_k(idx, *parts)


def _cat_nodes_body(p0, p1, p2, p3, out_ref):
    out_ref[...] = jnp.concatenate(
        [p0[...], p1[...], p2[...], p3[...]], axis=1)


def _finalize_nodes(parts, b, e):
    blk = 1024
    return pl.pallas_call(
        _cat_nodes_body,
        grid=(b // blk,),
        in_specs=[pl.BlockSpec((blk, 128), lambda i: (i, 0))] * 4,
        out_specs=pl.BlockSpec((blk, e), lambda i: (i, 0)),
        out_shape=jax.ShapeDtypeStruct((b, e), jnp.float32),
        compiler_params=pltpu.CompilerParams(
            dimension_semantics=("parallel",)),
    )(*parts)


def _cat_neigh_body(p0, p1, p2, p3, out_ref):
    cat = jnp.concatenate([p0[...], p1[...], p2[...], p3[...]], axis=1)
    out_ref[...] = cat.reshape(out_ref.shape)


def _finalize_neigh(parts, b, k, e):
    blk3 = 256
    return pl.pallas_call(
        _cat_neigh_body,
        grid=(b // blk3,),
        in_specs=[pl.BlockSpec((blk3 * k, 128), lambda i: (i, 0))] * 4,
        out_specs=pl.BlockSpec((blk3, k, e), lambda i: (i, 0, 0)),
        out_shape=jax.ShapeDtypeStruct((b, k, e), jnp.float32),
        compiler_params=pltpu.CompilerParams(
            dimension_semantics=("parallel",)),
    )(*parts)


def _mask_body(shift_ref, seq_ref, mask_ref):
    pos = lax.broadcasted_iota(jnp.int32, mask_ref.shape, 1) + 1 + shift_ref[0]
    mask_ref[...] = (pos > seq_ref[...]).astype(jnp.int8)


def _build_mask(seq2d, shift, k):
    bk = seq2d.shape[0]
    return pl.pallas_call(
        _mask_body,
        in_specs=[
            pl.BlockSpec(memory_space=pltpu.SMEM),
            pl.BlockSpec((bk, 1), lambda: (0, 0)),
        ],
        out_specs=pl.BlockSpec((bk, k), lambda: (0, 0)),
        out_shape=jax.ShapeDtypeStruct((bk, k), jnp.int8),
    )(shift, seq2d)


def kernel(nodes, neigh_idx, seq_length, num_sample, features_table, W_in, b_in):
    b, k = neigh_idx.shape
    n, d = features_table.shape
    e = W_in.shape[1]

    shift = (jnp.asarray(num_sample, jnp.int32) - k).reshape(1)
    seq2d = seq_length.astype(jnp.int32).reshape(b, 1)
    b2d = b_in.reshape(1, e)

    parts = _project_table(features_table, W_in, b2d)
    mask_i8 = _build_mask(seq2d, shift, k)

    try:
        info = plsc.get_sparse_core_info()
        nc, ns = info.num_cores, info.num_subcores
    except (RuntimeError, ValueError):
        nc, ns = 2, 16

    nodes_i = nodes.astype(jnp.int32)
    neigh_i = neigh_idx.astype(jnp.int32).reshape(-1)

    node_parts = _sc_gather4(parts, nodes_i, b, nc, ns)
    neigh_parts = _sc_gather4(parts, neigh_i, b * k, nc, ns)

    nodes_emb = _finalize_nodes(node_parts, b, e)
    neighs_emb = _finalize_neigh(neigh_parts, b, k, e)

    return (nodes_emb, neighs_emb, mask_i8.astype(jnp.bool_))
